# baseline (speedup 1.0000x reference)
import functools

import jax
import jax.numpy as jnp
from jax import lax
from jax.experimental import pallas as pl
from jax.experimental.pallas import tpu as pltpu
from jax.experimental.pallas import tpu_sc as plsc

VOCAB = 1000000
TEXT_DIM = 64
ENC_DIM = 128
BATCH = 4096
SEQ = 200

NT = BATCH * SEQ
NC = 2
NS = 16
NW = NC * NS
B_PER_W = NT // NW
CHUNK = 128
NCHUNK = B_PER_W // CHUNK

B2 = 2048


def _gather_body(idx_hbm, tab_hbm, out_hbm, idx_v, b0, b1, b2, b3, sem_g, sem_w):
    wid = lax.axis_index("s") * NC + lax.axis_index("c")
    base = wid * B_PER_W
    pltpu.sync_copy(idx_hbm.at[wid], idx_v)

    def fire_g(j, buf):
        pltpu.async_copy(tab_hbm.at[idx_v.at[j]], buf, sem_g)

    def wait_g(buf):
        pltpu.make_async_copy(tab_hbm.at[idx_v.at[0]], buf, sem_g).wait()

    def fire_w(j, buf):
        pltpu.async_copy(buf, out_hbm.at[pl.ds(base + j * CHUNK, CHUNK)], sem_w)

    def wait_w(buf):
        pltpu.make_async_copy(buf, out_hbm.at[pl.ds(base, CHUNK)], sem_w).wait()

    fire_g(0, b0)
    fire_g(1, b1)

    def body(g, carry):
        j0 = g * 4
        wait_g(b0)
        fire_w(j0, b0)

        @pl.when(g > 0)
        def _():
            wait_w(b2)
        fire_g(j0 + 2, b2)

        wait_g(b1)
        fire_w(j0 + 1, b1)

        @pl.when(g > 0)
        def _():
            wait_w(b3)
        fire_g(j0 + 3, b3)

        wait_g(b2)
        fire_w(j0 + 2, b2)
        wait_w(b0)

        @pl.when(j0 + 4 < NCHUNK)
        def _():
            fire_g(j0 + 4, b0)

        wait_g(b3)
        fire_w(j0 + 3, b3)
        wait_w(b1)

        @pl.when(j0 + 5 < NCHUNK)
        def _():
            fire_g(j0 + 5, b1)

        return carry

    lax.fori_loop(0, NCHUNK // 4, body, 0)
    wait_w(b2)
    wait_w(b3)


_gather = functools.partial(
    pl.kernel,
    mesh=plsc.VectorSubcoreMesh(core_axis_name="c", subcore_axis_name="s"),
    compiler_params=pltpu.CompilerParams(use_tc_tiling_on_sc=False),
    out_type=jax.ShapeDtypeStruct((NT, TEXT_DIM), jnp.float32),
    scratch_types=[
        pltpu.VMEM((NCHUNK, CHUNK), jnp.int32),
        pltpu.VMEM((CHUNK, TEXT_DIM), jnp.float32),
        pltpu.VMEM((CHUNK, TEXT_DIM), jnp.float32),
        pltpu.VMEM((CHUNK, TEXT_DIM), jnp.float32),
        pltpu.VMEM((CHUNK, TEXT_DIM), jnp.float32),
        pltpu.SemaphoreType.DMA,
        pltpu.SemaphoreType.DMA,
    ],
)(_gather_body)


def _mm_body(e2_ref, w_ref, b_ref, out_ref):
    blk = e2_ref[...]
    a = lax.dot_general(
        blk[:, :TEXT_DIM], w_ref[...],
        dimension_numbers=(((1,), (0,)), ((), ())),
        preferred_element_type=jnp.float32,
    ) + b_ref[...]
    c = lax.dot_general(
        blk[:, TEXT_DIM:], w_ref[...],
        dimension_numbers=(((1,), (0,)), ((), ())),
        preferred_element_type=jnp.float32,
    ) + b_ref[...]
    out_ref[...] = jnp.concatenate(
        [a[:, None, :], c[:, None, :]], axis=1
    ).reshape(2 * B2, ENC_DIM)


_project = pl.pallas_call(
    _mm_body,
    grid=(NT // 2 // B2,),
    in_specs=[
        pl.BlockSpec((B2, 2 * TEXT_DIM), lambda i: (i, 0)),
        pl.BlockSpec((TEXT_DIM, ENC_DIM), lambda i: (0, 0)),
        pl.BlockSpec((1, ENC_DIM), lambda i: (0, 0)),
    ],
    out_specs=pl.BlockSpec((2 * B2, ENC_DIM), lambda i: (i, 0)),
    out_shape=jax.ShapeDtypeStruct((NT, ENC_DIM), jnp.float32),
)


@jax.jit
def kernel(x, table, W, b):
    idx = x.reshape(NW, NCHUNK, CHUNK)
    emb = _gather(idx, table)
    e2 = emb.reshape(NT // 2, 2 * TEXT_DIM)
    out = _project(e2, W, b.reshape(1, ENC_DIM))
    return out.reshape(BATCH, SEQ, ENC_DIM)


# speedup vs baseline: 2.1099x; 2.1099x over previous
import functools

import jax
import jax.numpy as jnp
from jax import lax
from jax.experimental import pallas as pl
from jax.experimental.pallas import tpu as pltpu
from jax.experimental.pallas import tpu_sc as plsc

VOCAB = 1000000
TEXT_DIM = 64
ENC_DIM = 128
BATCH = 4096
SEQ = 200

NT = BATCH * SEQ
NC = 2
NS = 16
NW = NC * NS
B_PER_W = NT // NW
CHUNK = 128
NCHUNK = B_PER_W // CHUNK

V_BLK = 32768


def _proj_body(tT_ref, w_ref, b_ref, out_ref):
    out_ref[...] = (
        lax.dot_general(
            tT_ref[...], w_ref[...],
            dimension_numbers=(((0,), (0,)), ((), ())),
            preferred_element_type=jnp.float32,
        )
        + b_ref[...]
    )


_project = pl.pallas_call(
    _proj_body,
    grid=(pl.cdiv(VOCAB, V_BLK),),
    in_specs=[
        pl.BlockSpec((TEXT_DIM, V_BLK), lambda i: (0, i)),
        pl.BlockSpec((TEXT_DIM, ENC_DIM), lambda i: (0, 0)),
        pl.BlockSpec((1, ENC_DIM), lambda i: (0, 0)),
    ],
    out_specs=pl.BlockSpec((V_BLK, ENC_DIM), lambda i: (i, 0)),
    out_shape=jax.ShapeDtypeStruct((VOCAB, ENC_DIM), jnp.float32),
)


def _gather_body(idx_hbm, pt_hbm, out_hbm, idx_v,
                 b0, b1, b2, b3, b4, b5, sem_g, sem_w):
    wid = lax.axis_index("s") * NC + lax.axis_index("c")
    base = wid * B_PER_W
    pltpu.sync_copy(idx_hbm.at[wid], idx_v)

    bufs = (b0, b1, b2, b3, b4, b5)

    def fire_g(j, buf):
        pltpu.async_copy(pt_hbm.at[idx_v.at[j]], buf, sem_g)

    def wait_g(buf):
        pltpu.make_async_copy(pt_hbm.at[idx_v.at[0]], buf, sem_g).wait()

    def fire_w(j, buf):
        pltpu.async_copy(buf, out_hbm.at[pl.ds(base + j * CHUNK, CHUNK)], sem_w)

    def wait_w(buf):
        pltpu.make_async_copy(buf, out_hbm.at[pl.ds(base, CHUNK)], sem_w).wait()

    fire_g(0, b0)
    fire_g(1, b1)
    fire_g(2, b2)
    fire_g(3, b3)

    def body(g, carry):
        j0 = g * 6
        for t in range(6):
            j = j0 + t
            buf = bufs[t]
            nbuf = bufs[(t + 4) % 6]
            wait_g(buf)
            fire_w(j, buf)

            @pl.when(j + 4 < NCHUNK)
            def _():
                @pl.when(j >= 2)
                def _():
                    wait_w(nbuf)
                fire_g(j + 4, nbuf)

        return carry

    lax.fori_loop(0, NCHUNK // 6, body, 0)
    wait_g(bufs[(NCHUNK - 2) % 6])
    fire_w(NCHUNK - 2, bufs[(NCHUNK - 2) % 6])
    wait_g(bufs[(NCHUNK - 1) % 6])
    fire_w(NCHUNK - 1, bufs[(NCHUNK - 1) % 6])
    for k in range(NCHUNK - 6, NCHUNK):
        wait_w(bufs[k % 6])


_gather = functools.partial(
    pl.kernel,
    mesh=plsc.VectorSubcoreMesh(core_axis_name="c", subcore_axis_name="s"),
    compiler_params=pltpu.CompilerParams(use_tc_tiling_on_sc=False),
    out_type=jax.ShapeDtypeStruct((NT, ENC_DIM), jnp.float32),
    scratch_types=[
        pltpu.VMEM((NCHUNK, CHUNK), jnp.int32),
        pltpu.VMEM((CHUNK, ENC_DIM), jnp.float32),
        pltpu.VMEM((CHUNK, ENC_DIM), jnp.float32),
        pltpu.VMEM((CHUNK, ENC_DIM), jnp.float32),
        pltpu.VMEM((CHUNK, ENC_DIM), jnp.float32),
        pltpu.VMEM((CHUNK, ENC_DIM), jnp.float32),
        pltpu.VMEM((CHUNK, ENC_DIM), jnp.float32),
        pltpu.SemaphoreType.DMA,
        pltpu.SemaphoreType.DMA,
    ],
)(_gather_body)


@jax.jit
def kernel(x, table, W, b):
    pt = _project(table.T, W, b.reshape(1, ENC_DIM))
    idx = x.reshape(NW, NCHUNK, CHUNK)
    out = _gather(idx, pt)
    return out.reshape(BATCH, SEQ, ENC_DIM)
